# Initial kernel scaffold; baseline (speedup 1.0000x reference)
import jax
import jax.numpy as jnp
from jax.experimental import pallas as pl


def kernel(x, table):
    raise NotImplementedError("write your pallas kernel here")


# speedup vs baseline: 1.8698x; 1.8698x over previous
import functools

import jax
import jax.numpy as jnp
from jax import lax
from jax.experimental import pallas as pl
from jax.experimental.pallas import tpu as pltpu
from jax.experimental.pallas import tpu_sc as plsc

_D = 512
_B = 4096 * 50
_NC, _NS = 2, 16
_NW = _NC * _NS
_BPW = _B // _NW
_C = 80
_NCH = _BPW // _C
_NBUF = 2


def _make_gather():
  mesh = plsc.VectorSubcoreMesh(core_axis_name="c", subcore_axis_name="s")
  scratch = [pltpu.VMEM((_NCH, _C), jnp.int32)]
  scratch += [pltpu.VMEM((_C, _D), jnp.float32) for _ in range(_NBUF)]
  scratch += [pltpu.SemaphoreType.DMA for _ in range(2 * _NBUF)]

  @functools.partial(
      pl.kernel,
      mesh=mesh,
      out_type=jax.ShapeDtypeStruct((_B, _D), jnp.float32),
      scratch_types=scratch,
  )
  def gather_kernel(idx_hbm, table_hbm, out_hbm, idx_v, *rest):
    bufs = rest[:_NBUF]
    in_sems = rest[_NBUF:2 * _NBUF]
    out_sems = rest[2 * _NBUF:]
    wid = lax.axis_index("s") * _NC + lax.axis_index("c")
    row0 = wid * _BPW

    pltpu.sync_copy(idx_hbm.at[pl.ds(wid * _NCH, _NCH)], idx_v)

    def start_gather(c, b):
      pltpu.async_copy(table_hbm.at[idx_v.at[c]], bufs[b], in_sems[b])

    def wait_gather(b):
      pltpu.make_async_copy(
          table_hbm.at[idx_v.at[0]], bufs[b], in_sems[b]).wait()

    def start_scatter(c, b):
      pltpu.async_copy(
          bufs[b], out_hbm.at[pl.ds(row0 + c * _C, _C)], out_sems[b])

    def wait_scatter(b):
      pltpu.make_async_copy(
          bufs[b], out_hbm.at[pl.ds(row0, _C)], out_sems[b]).wait()

    for b in range(_NBUF):
      start_gather(b, b)

    def body(o, carry):
      for b in range(_NBUF):
        c = o * _NBUF + b
        wait_gather(b)
        start_scatter(c, b)
        wait_scatter(b)
        start_gather(c + _NBUF, b)
      return carry

    lax.fori_loop(0, _NCH // _NBUF - 1, body, 0)

    for b in range(_NBUF):
      wait_gather(b)
      start_scatter(_NCH - _NBUF + b, b)
    for b in range(_NBUF):
      wait_scatter(b)

  return gather_kernel


_gather = _make_gather()


def kernel(x, table):
  idx = x.reshape(_NW * _NCH, _C).astype(jnp.int32)
  out = _gather(idx, table)
  return out.reshape(x.shape[0], x.shape[1], _D)
